# Initial kernel scaffold; baseline (speedup 1.0000x reference)
import jax
import jax.numpy as jnp
from jax.experimental import pallas as pl


def kernel(obj_logits, vr, boxes_per_cls, W, b):
    raise NotImplementedError("write your pallas kernel here")


# speedup vs baseline: 4.1319x; 4.1319x over previous
import functools

import jax
import jax.numpy as jnp
from jax.experimental import pallas as pl

NMS_THRESH = 0.3
N = 1000
NP = 1024
C = 151


def _nms_argmax_kernel(p_ref, best_ref, pred_ref):
    c = pl.program_id(0)

    @pl.when(c == 0)
    def _init():
        best_ref[...] = jnp.full(best_ref.shape, -1.0, jnp.float32)
        pred_ref[...] = jnp.full(pred_ref.shape, 1, jnp.int32)

    p = p_ref[0]
    x1r = p[0:1, :]
    y1r = p[1:2, :]
    x2r = p[2:3, :]
    y2r = p[3:4, :]
    sr = p[4:5, :]
    ar = (x2r - x1r + 1.0) * (y2r - y1r + 1.0)

    x1c = x1r.T
    y1c = y1r.T
    x2c = x2r.T
    y2c = y2r.T
    sc = sr.T
    ac = ar.T

    xx1 = jnp.maximum(x1c, x1r)
    yy1 = jnp.maximum(y1c, y1r)
    xx2 = jnp.minimum(x2c, x2r)
    yy2 = jnp.minimum(y2c, y2r)
    w = jnp.maximum(0.0, xx2 - xx1 + 1.0)
    h = jnp.maximum(0.0, yy2 - yy1 + 1.0)
    inter = w * h
    iou = inter / (ac + ar - inter)

    rowi = jax.lax.broadcasted_iota(jnp.int32, (NP, NP), 0)
    coli = jax.lax.broadcasted_iota(jnp.int32, (NP, NP), 1)
    dom = (sc > sr) | ((sc == sr) & (rowi < coli))
    m = ((iou > NMS_THRESH) & dom).astype(jnp.float32)

    def body(carry):
        k, _ = carry
        cnt = jnp.dot(k, m, preferred_element_type=jnp.float32)
        new = (cnt == 0.0).astype(jnp.float32)
        return new, jnp.any(new != k)

    def cond(carry):
        return carry[1]

    k0 = jnp.ones((1, NP), jnp.float32)
    keep, _ = jax.lax.while_loop(cond, body, (k0, jnp.bool_(True)))

    val = keep * sr
    best = best_ref[0:1, :]
    upd = val > best
    best_ref[0:1, :] = jnp.where(upd, val, best)
    cls = jnp.full((1, NP), c + 1, jnp.int32)
    pred_ref[0:1, :] = jnp.where(upd, cls, pred_ref[0:1, :])


def _relhead_kernel(vr_ref, w_ref, b_ref, out_ref):
    acc = jax.lax.dot_general(
        vr_ref[...], w_ref[...],
        dimension_numbers=(((1,), (1,)), ((), ())),
        preferred_element_type=jnp.float32,
    )
    out_ref[...] = acc + b_ref[...]


@jax.jit
def kernel(obj_logits, vr, boxes_per_cls, W, b):
    probs = jax.nn.softmax(obj_logits, axis=1)

    bT = jnp.transpose(boxes_per_cls, (1, 2, 0))
    bT = jnp.pad(bT, ((0, 0), (0, 0), (0, NP - N)))
    sT = jnp.pad(probs.T[:, None, :], ((0, 0), (0, 0), (0, NP - N)),
                 constant_values=-1.0)
    pad = jnp.zeros((C, 3, NP), jnp.float32)
    packed = jnp.concatenate([bT, sT, pad], axis=1)

    best, preds = pl.pallas_call(
        _nms_argmax_kernel,
        grid=(C - 1,),
        in_specs=[pl.BlockSpec((1, 8, NP), lambda c: (c + 1, 0, 0))],
        out_specs=[pl.BlockSpec((8, NP), lambda c: (0, 0)),
                   pl.BlockSpec((8, NP), lambda c: (0, 0))],
        out_shape=[jax.ShapeDtypeStruct((8, NP), jnp.float32),
                   jax.ShapeDtypeStruct((8, NP), jnp.int32)],
    )(packed)
    obj_preds = preds[0, :N]

    RB = 400
    rel_dists = pl.pallas_call(
        _relhead_kernel,
        grid=(vr.shape[0] // RB,),
        in_specs=[pl.BlockSpec((RB, vr.shape[1]), lambda i: (i, 0)),
                  pl.BlockSpec(W.shape, lambda i: (0, 0)),
                  pl.BlockSpec((1, W.shape[0]), lambda i: (0, 0))],
        out_specs=pl.BlockSpec((RB, W.shape[0]), lambda i: (i, 0)),
        out_shape=jax.ShapeDtypeStruct((vr.shape[0], W.shape[0]), jnp.float32),
    )(vr, W, b.reshape(1, -1))

    return (obj_logits, obj_preds, rel_dists)
